# Initial kernel scaffold; baseline (speedup 1.0000x reference)
import jax
import jax.numpy as jnp
from jax.experimental import pallas as pl


def kernel(z, embedding):
    raise NotImplementedError("write your pallas kernel here")


# speedup vs baseline: 7.2721x; 7.2721x over previous
import jax
import jax.numpy as jnp
from jax.experimental import pallas as pl
from jax.experimental.pallas import tpu as pltpu

_N = 8
_K = 1024
_E = 64
_DEPTH = 2
_BETA = 0.25
_B = 4096
_D = 512
_TB = 512
_PAIR = 2


def _quantize(zt, e_n, e_sq, e_hi, e_mid, e_lo):
    tb = zt.shape[0]
    iota_i = jax.lax.broadcasted_iota(jnp.int32, (tb, _K), 1)
    r = zt
    zq_n = jnp.zeros_like(zt)
    oh = None
    idx_t = None
    for d in range(_DEPTH):
        s_t = e_sq - 2.0 * jax.lax.dot_general(
            e_n, r, (((1,), (1,)), ((), ())),
            preferred_element_type=jnp.float32,
            precision=jax.lax.Precision.HIGHEST)
        idx_t = jnp.argmin(s_t, axis=0, keepdims=True)
        idx2 = jax.lax.transpose(idx_t, (1, 0))
        oh32 = jnp.where(iota_i == idx2, jnp.float32(1),
                         jnp.float32(0))
        oh_bf = oh32.astype(jnp.bfloat16)
        if d == _DEPTH - 1:
            oh = oh32
        q = jax.lax.dot_general(
            oh_bf, e_hi, (((1,), (0,)), ((), ())),
            preferred_element_type=jnp.float32)
        r = r - q
        zq_n = zq_n + q
    zq_n = zq_n * 0.5
    diff = zq_n - zt
    return zt + diff, oh, idx_t, diff


def _vq_kernel(z_ref, emb_ref, zq_ref, oh_ref, idx_ref, loss_ref,
               esq_ref, ehi_ref, emid_ref, elo_ref):
    i = pl.program_id(0)
    g = pl.program_id(1)

    @pl.when((i == 0) & (g == 0))
    def _init():
        loss_ref[0, 0] = 0.0

    @pl.when(i == 0)
    def _cache():
        for p in range(_PAIR):
            e_n = emb_ref[p]
            np_ = g * _PAIR + p
            esq_ref[np_] = jnp.sum(e_n * e_n, axis=1, keepdims=True)
            e_hi = e_n.astype(jnp.bfloat16)
            r1 = e_n - e_hi.astype(jnp.float32)
            e_mid = r1.astype(jnp.bfloat16)
            e_lo = (r1 - e_mid.astype(jnp.float32)).astype(jnp.bfloat16)
            ehi_ref[np_] = e_hi
            emid_ref[np_] = e_mid
            elo_ref[np_] = e_lo

    loss_part = jnp.float32(0.0)
    for p in range(_PAIR):
        np_ = g * _PAIR + p
        zq_st, oh, idx_t, diff = _quantize(
            z_ref[p], emb_ref[p], esq_ref[np_], ehi_ref[np_],
            emid_ref[np_], elo_ref[np_])
        zq_ref[p] = zq_st
        idx_ref[p] = idx_t
        for gg in range(_N // _PAIR):
            @pl.when(g == gg)
            def _store(nn=gg * _PAIR + p, oh=oh):
                oh_ref[:, nn * _K:(nn + 1) * _K] = oh
        loss_part = loss_part + jnp.sum(diff * diff)
    loss_ref[0, 0] += loss_part * ((1.0 + _BETA) / (_B * _D))


def kernel(z, embedding):
    zt3 = z.reshape(_B, _N, _E).swapaxes(0, 1)
    grid = (_B // _TB, _N // _PAIR)
    zq, one_hot, idx, loss = pl.pallas_call(
        _vq_kernel,
        grid=grid,
        in_specs=[
            pl.BlockSpec((_PAIR, _TB, _E), lambda i, g: (g, i, 0)),
            pl.BlockSpec((_PAIR, _K, _E), lambda i, g: (g, 0, 0)),
        ],
        out_specs=(
            pl.BlockSpec((_PAIR, _TB, _E), lambda i, g: (g, i, 0)),
            pl.BlockSpec((_TB, _N * _K), lambda i, g: (i, 0)),
            pl.BlockSpec((_PAIR, 1, _TB), lambda i, g: (g, 0, i)),
            pl.BlockSpec((1, 1), lambda i, g: (0, 0),
                         memory_space=pltpu.SMEM),
        ),
        out_shape=(
            jax.ShapeDtypeStruct((_N, _B, _E), jnp.float32),
            jax.ShapeDtypeStruct((_B, _N * _K), jnp.float32),
            jax.ShapeDtypeStruct((_N, 1, _B), jnp.int32),
            jax.ShapeDtypeStruct((1, 1), jnp.float32),
        ),
        scratch_shapes=[
            pltpu.VMEM((_N, _K, 1), jnp.float32),
            pltpu.VMEM((_N, _K, _E), jnp.bfloat16),
            pltpu.VMEM((_N, _K, _E), jnp.bfloat16),
            pltpu.VMEM((_N, _K, _E), jnp.bfloat16),
        ],
    )(zt3, embedding)
    perplexity = jnp.zeros((), jnp.float32)
    zq_out = zq.swapaxes(0, 1).reshape(_B, _D)
    idx_out = idx[:, 0, :].swapaxes(0, 1)[..., None]
    return (loss[0, 0], zq_out, perplexity,
            one_hot.reshape(_B, _N, _K), idx_out)


# speedup vs baseline: 7.7713x; 1.0686x over previous
import jax
import jax.numpy as jnp
from jax.experimental import pallas as pl
from jax.experimental.pallas import tpu as pltpu

_N = 8
_K = 1024
_E = 64
_DEPTH = 2
_BETA = 0.25
_B = 4096
_D = 512
_TB = 512
_PAIR = 2


def _quantize(zt, e_n, e_sq, e_hi):
    tb = zt.shape[0]
    iota_i = jax.lax.broadcasted_iota(jnp.int32, (tb, _K), 1)
    r = zt
    zq_n = jnp.zeros_like(zt)
    oh = None
    idx_t = None
    for d in range(_DEPTH):
        s_t = e_sq - 2.0 * jax.lax.dot_general(
            e_n, r, (((1,), (1,)), ((), ())),
            preferred_element_type=jnp.float32,
            precision=jax.lax.Precision.HIGHEST)
        idx_t = jnp.argmin(s_t, axis=0, keepdims=True)
        idx2 = jax.lax.transpose(idx_t, (1, 0))
        oh32 = jnp.where(iota_i == idx2, jnp.float32(1),
                         jnp.float32(0))
        oh_bf = oh32.astype(jnp.bfloat16)
        if d == _DEPTH - 1:
            oh = oh32
        q = jax.lax.dot_general(
            oh_bf, e_hi, (((1,), (0,)), ((), ())),
            preferred_element_type=jnp.float32)
        r = r - q
        zq_n = zq_n + q
    zq_n = zq_n * 0.5
    diff = zq_n - zt
    return zt + diff, oh, idx_t, diff


def _vq_kernel(z_ref, emb_ref, zq_ref, oh_ref, idx_ref, loss_ref,
               esq_ref, ehi_ref):
    i = pl.program_id(0)
    g = pl.program_id(1)

    @pl.when((i == 0) & (g == 0))
    def _init():
        loss_ref[0, 0] = 0.0

    @pl.when(i == 0)
    def _cache():
        for p in range(_PAIR):
            e_n = emb_ref[p]
            np_ = g * _PAIR + p
            esq_ref[np_] = jnp.sum(e_n * e_n, axis=1, keepdims=True)
            ehi_ref[np_] = e_n.astype(jnp.bfloat16)

    loss_part = jnp.float32(0.0)
    for p in range(_PAIR):
        np_ = g * _PAIR + p
        zt = jnp.zeros((_TB, _E), jnp.float32)
        for gg in range(_N // _PAIR):
            nn = gg * _PAIR + p
            zt = jnp.where(g == gg,
                           z_ref[:, nn * _E:(nn + 1) * _E], zt)
        zq_st, oh, idx_t, diff = _quantize(
            zt, emb_ref[p], esq_ref[np_], ehi_ref[np_])
        idx_ref[p] = idx_t
        for gg in range(_N // _PAIR):
            @pl.when(g == gg)
            def _store(nn=gg * _PAIR + p, oh=oh, zq_st=zq_st):
                oh_ref[:, nn * _K:(nn + 1) * _K] = oh
                zq_ref[:, nn * _E:(nn + 1) * _E] = zq_st
        loss_part = loss_part + jnp.sum(diff * diff)
    loss_ref[0, 0] += loss_part * ((1.0 + _BETA) / (_B * _D))


def kernel(z, embedding):
    grid = (_B // _TB, _N // _PAIR)
    zq, one_hot, idx, loss = pl.pallas_call(
        _vq_kernel,
        grid=grid,
        in_specs=[
            pl.BlockSpec((_TB, _D), lambda i, g: (i, 0)),
            pl.BlockSpec((_PAIR, _K, _E), lambda i, g: (g, 0, 0)),
        ],
        out_specs=(
            pl.BlockSpec((_TB, _D), lambda i, g: (i, 0)),
            pl.BlockSpec((_TB, _N * _K), lambda i, g: (i, 0)),
            pl.BlockSpec((_PAIR, 1, _TB), lambda i, g: (g, 0, i)),
            pl.BlockSpec((1, 1), lambda i, g: (0, 0),
                         memory_space=pltpu.SMEM),
        ),
        out_shape=(
            jax.ShapeDtypeStruct((_B, _D), jnp.float32),
            jax.ShapeDtypeStruct((_B, _N * _K), jnp.float32),
            jax.ShapeDtypeStruct((_N, 1, _B), jnp.int32),
            jax.ShapeDtypeStruct((1, 1), jnp.float32),
        ),
        scratch_shapes=[
            pltpu.VMEM((_N, _K, 1), jnp.float32),
            pltpu.VMEM((_N, _K, _E), jnp.bfloat16),
        ],
    )(z, embedding)
    perplexity = jnp.zeros((), jnp.float32)
    idx_out = idx[:, 0, :].swapaxes(0, 1)[..., None]
    return (loss[0, 0], zq, perplexity,
            one_hot.reshape(_B, _N, _K), idx_out)


# speedup vs baseline: 9.0844x; 1.1690x over previous
import jax
import jax.numpy as jnp
from jax.experimental import pallas as pl
from jax.experimental.pallas import tpu as pltpu

_N = 8
_K = 1024
_E = 64
_DEPTH = 2
_BETA = 0.25
_B = 4096
_D = 512
_TB = 512
_PAIR = 2


def _quantize(zt, e_n, e_sq, e_hi):
    tb = zt.shape[0]
    iota_i = jax.lax.broadcasted_iota(jnp.int32, (tb, _K), 1)
    r = zt
    zq_n = jnp.zeros_like(zt)
    oh = None
    idx_t = None
    for d in range(_DEPTH):
        s_t = e_sq - 2.0 * jax.lax.dot_general(
            e_n, r, (((1,), (1,)), ((), ())),
            preferred_element_type=jnp.float32,
            precision=jax.lax.Precision.HIGHEST)
        idx_t = jnp.argmin(s_t, axis=0, keepdims=True)
        idx2 = jax.lax.transpose(idx_t, (1, 0))
        oh32 = jnp.where(iota_i == idx2, jnp.float32(1),
                         jnp.float32(0))
        oh_bf = oh32.astype(jnp.bfloat16)
        if d == _DEPTH - 1:
            oh = oh32
        q = jax.lax.dot_general(
            oh_bf, e_hi, (((1,), (0,)), ((), ())),
            preferred_element_type=jnp.float32)
        r = r - q
        zq_n = zq_n + q
    zq_n = zq_n * 0.5
    diff = zq_n - zt
    return zt + diff, oh, idx_t, diff


def _vq_kernel(z_ref, emb_ref, zq_ref, oh_ref, idx_ref, loss_ref,
               esq_ref, ehi_ref):
    i = pl.program_id(0)
    g = pl.program_id(1)

    @pl.when((i == 0) & (g == 0))
    def _init():
        loss_ref[0, 0] = 0.0

    @pl.when(i == 0)
    def _cache():
        for p in range(_PAIR):
            e_n = emb_ref[p]
            np_ = g * _PAIR + p
            esq_ref[np_] = jnp.sum(e_n * e_n, axis=1, keepdims=True)
            ehi_ref[np_] = e_n.astype(jnp.bfloat16)

    loss_part = jnp.float32(0.0)
    for p in range(_PAIR):
        np_ = g * _PAIR + p
        zt = jnp.zeros((_TB, _E), jnp.float32)
        for gg in range(_N // _PAIR):
            nn = gg * _PAIR + p
            zt = jnp.where(g == gg,
                           z_ref[:, nn * _E:(nn + 1) * _E], zt)
        zq_st, oh, idx_t, diff = _quantize(
            zt, emb_ref[p], esq_ref[np_], ehi_ref[np_])
        idx_ref[p] = idx_t
        for gg in range(_N // _PAIR):
            @pl.when(g == gg)
            def _store(nn=gg * _PAIR + p, oh=oh, zq_st=zq_st):
                oh_ref[:, nn, :] = oh
                zq_ref[:, nn * _E:(nn + 1) * _E] = zq_st
        loss_part = loss_part + jnp.sum(diff * diff)
    loss_ref[0, 0] += loss_part * ((1.0 + _BETA) / (_B * _D))


def kernel(z, embedding):
    grid = (_B // _TB, _N // _PAIR)
    zq, one_hot, idx, loss = pl.pallas_call(
        _vq_kernel,
        grid=grid,
        in_specs=[
            pl.BlockSpec((_TB, _D), lambda i, g: (i, 0)),
            pl.BlockSpec((_PAIR, _K, _E), lambda i, g: (g, 0, 0)),
        ],
        out_specs=(
            pl.BlockSpec((_TB, _D), lambda i, g: (i, 0)),
            pl.BlockSpec((_TB, _N, _K), lambda i, g: (i, 0, 0)),
            pl.BlockSpec((_PAIR, 1, _TB), lambda i, g: (g, 0, i)),
            pl.BlockSpec((1, 1), lambda i, g: (0, 0),
                         memory_space=pltpu.SMEM),
        ),
        out_shape=(
            jax.ShapeDtypeStruct((_B, _D), jnp.float32),
            jax.ShapeDtypeStruct((_B, _N, _K), jnp.float32),
            jax.ShapeDtypeStruct((_N, 1, _B), jnp.int32),
            jax.ShapeDtypeStruct((1, 1), jnp.float32),
        ),
        scratch_shapes=[
            pltpu.VMEM((_N, _K, 1), jnp.float32),
            pltpu.VMEM((_N, _K, _E), jnp.bfloat16),
        ],
    )(z, embedding)
    perplexity = jnp.zeros((), jnp.float32)
    idx_out = idx[:, 0, :].swapaxes(0, 1)[..., None]
    return (loss[0, 0], zq, perplexity, one_hot, idx_out)
